# baseline (speedup 1.0000x reference)
import functools

import jax
import jax.numpy as jnp
from jax import lax
from jax.experimental import pallas as pl
from jax.experimental.pallas import tpu as pltpu
from jax.experimental.pallas import tpu_sc as plsc

B = 16384
F = 26
V = 100000
D = 32
L_ID = 8
ATT = D - L_ID
R = B * F

NC = 2
NS = 16
NW = NC * NS
RW = R // NW
CH = 1664
NCH = RW // CH
G = 128
NG = CH // G
LANES = 16


def _body(
    in_hbm, tab_hbm, id_hbm, pat_hbm, out_hbm, pat_v, in_v, idx_v, idc_v, rows_v, sem
):
    cid = lax.axis_index("c")
    sid = lax.axis_index("s")
    wid = cid * NS + sid
    r0 = wid * RW

    for j in range(CH // F):
        pltpu.sync_copy(id_hbm, idc_v.at[pl.ds(F * j, F)])

    pltpu.sync_copy(pat_hbm, pat_v)

    for t in range(NCH):
        base = r0 + t * CH
        pltpu.sync_copy(in_hbm.at[pl.ds(base, CH)], in_v)
        for k in range(CH // LANES):
            g, l = divmod(k, G // LANES)
            idx_v[g, pl.ds(LANES * l, LANES)] = (
                in_v[pl.ds(LANES * k, LANES)] + pat_v[pl.ds(LANES * k, LANES)]
            )
        handles = [
            pltpu.async_copy(
                tab_hbm.at[idx_v.at[g]],
                rows_v.at[pl.ds(G * g, G)],
                sem,
            )
            for g in range(NG)
        ]
        for h in handles:
            h.wait()
        pltpu.sync_copy(rows_v, out_hbm.at[pl.ds(base, CH), :, pl.ds(L_ID, ATT)])
        pltpu.sync_copy(idc_v, out_hbm.at[pl.ds(base, CH), :, pl.ds(0, L_ID)])


_sc_call = functools.partial(
    pl.kernel,
    out_type=jax.ShapeDtypeStruct((R, 1, D), jnp.float32),
    compiler_params=pltpu.CompilerParams(use_tc_tiling_on_sc=False),
    mesh=plsc.VectorSubcoreMesh(
        core_axis_name="c", subcore_axis_name="s", num_cores=NC, num_subcores=NS
    ),
    scratch_types=[
        pltpu.VMEM((CH,), jnp.int32),
        pltpu.VMEM((CH,), jnp.int32),
        pltpu.VMEM((NG, G), jnp.int32),
        pltpu.VMEM((CH, 1, L_ID), jnp.float32),
        pltpu.VMEM((CH, 1, ATT), jnp.float32),
        pltpu.SemaphoreType.DMA,
    ],
)(_body)


def kernel(inputs, tables, id_table):
    in_flat = inputs.reshape(R).astype(jnp.int32)
    tab_flat = tables.reshape(F * V, 1, ATT)
    id3 = id_table.reshape(F, 1, L_ID)
    pat = (jnp.arange(CH, dtype=jnp.int32) % F) * V
    out = _sc_call(in_flat, tab_flat, id3, pat)
    return out.reshape(B, F, D)


# speedup vs baseline: 6.8764x; 6.8764x over previous
import functools

import jax
import jax.numpy as jnp
from jax import lax
from jax.experimental import pallas as pl
from jax.experimental.pallas import tpu as pltpu
from jax.experimental.pallas import tpu_sc as plsc

B = 16384
F = 26
V = 100000
D = 32
L_ID = 8
ATT = D - L_ID
R = B * F

NC = 2
NS = 16
NW = NC * NS
RW = R // NW
CH = 1664
NCH = RW // CH
G = 128
NG = CH // G
LANES = 16


def _body(
    in_hbm, tab_hbm, id_hbm, pat_hbm, out_hbm, pat_v, in_v, idx_v, idc_v, rows_v, sem
):
    cid = lax.axis_index("c")
    sid = lax.axis_index("s")
    wid = cid * NS + sid
    r0 = wid * RW

    for j in range(CH // F):
        pltpu.sync_copy(id_hbm, idc_v.at[pl.ds(F * j, F)])

    pltpu.sync_copy(pat_hbm, pat_v)

    for t in range(NCH):
        base = r0 + t * CH
        pltpu.sync_copy(in_hbm.at[pl.ds(base, CH)], in_v)
        for k in range(CH // LANES):
            g, l = divmod(k, G // LANES)
            idx_v[g, pl.ds(LANES * l, LANES)] = (
                in_v[pl.ds(LANES * k, LANES)] + pat_v[pl.ds(LANES * k, LANES)]
            )
        handles = [
            pltpu.async_copy(
                tab_hbm.at[idx_v.at[g]],
                rows_v.at[pl.ds(G * g, G)],
                sem,
            )
            for g in range(NG)
        ]
        for h in handles:
            h.wait()
        pltpu.sync_copy(rows_v, out_hbm.at[pl.ds(base, CH), pl.ds(L_ID, ATT)])
        pltpu.sync_copy(idc_v, out_hbm.at[pl.ds(base, CH), pl.ds(0, L_ID)])


_sc_call = functools.partial(
    pl.kernel,
    out_type=jax.ShapeDtypeStruct((R, D), jnp.float32),
    compiler_params=pltpu.CompilerParams(use_tc_tiling_on_sc=False),
    mesh=plsc.VectorSubcoreMesh(
        core_axis_name="c", subcore_axis_name="s", num_cores=NC, num_subcores=NS
    ),
    scratch_types=[
        pltpu.VMEM((CH,), jnp.int32),
        pltpu.VMEM((CH,), jnp.int32),
        pltpu.VMEM((NG, G), jnp.int32),
        pltpu.VMEM((CH, L_ID), jnp.float32),
        pltpu.VMEM((CH, ATT), jnp.float32),
        pltpu.SemaphoreType.DMA,
    ],
)(_body)


def kernel(inputs, tables, id_table):
    in_flat = inputs.reshape(R).astype(jnp.int32)
    tab_flat = tables.reshape(F * V, ATT)
    id3 = id_table
    pat = (jnp.arange(CH, dtype=jnp.int32) % F) * V
    out = _sc_call(in_flat, tab_flat, id3, pat)
    return out.reshape(B, F, D)
